# baseline (speedup 1.0000x reference)
import jax
import jax.numpy as jnp
from jax import lax
from jax.experimental import pallas as pl
from jax.experimental.pallas import tpu as pltpu
from jax.experimental.pallas import tpu_sc as plsc

NU = 5000
NEV = 5000
D = 128
E = 320000
NC = 2
NS = 16
NW = NC * NS
CH = 128
CPT = 80
EP = NW * CPT * CH
NROW = EP // CH
K = 2
NB = CPT // K
R = 5120
RPT = R // NS
TRASH = 5000

_f32 = jnp.float32


def _make_sc_agg(with_deg):
  mesh = plsc.VectorSubcoreMesh(core_axis_name="c", subcore_axis_name="s")

  out_type = [
      jax.ShapeDtypeStruct((NC, R, D), _f32),
      jax.ShapeDtypeStruct((NC, R, D), _f32),
  ]
  if with_deg:
    out_type += [
        jax.ShapeDtypeStruct((NC, R, D), _f32),
        jax.ShapeDtypeStruct((NC, R, D), _f32),
    ]
  scratch = [
      pltpu.VMEM((2, K, CH), jnp.int32),
      pltpu.VMEM((2, K, CH), jnp.int32),
      pltpu.VMEM((CH, D), _f32),
      pltpu.VMEM((CH, D), _f32),
      pltpu.VMEM((CH, D), _f32),
      pltpu.VMEM((CH, D), _f32),
      pltpu.VMEM_SHARED((R, D), _f32),
      pltpu.SemaphoreType.DMA,
      pltpu.SemaphoreType.DMA,
      pltpu.SemaphoreType.DMA,
  ]

  def body(*refs):
    if with_deg:
      (he, hu, src2, dst2, zeros, ones,
       aggu_o, agge_o, degu_o, dege_o,
       gib, sib, r0, r1, r2, r3, acc, sg, ss, si) = refs
    else:
      (he, hu, src2, dst2, zeros,
       aggu_o, agge_o,
       gib, sib, r0, r1, r2, r3, acc, sg, ss, si) = refs

    rows = (r0, r1, r2, r3)
    c = lax.axis_index("c")
    s = lax.axis_index("s")
    wid = s * NC + c
    row_base = wid * CPT
    slab = pl.ds(s * RPT, RPT)

    def init_acc():
      pltpu.sync_copy(zeros.at[slab], acc.at[slab])
      plsc.subcore_barrier()

    def dump_acc(out_ref):
      plsc.subcore_barrier()
      pltpu.sync_copy(acc.at[slab], out_ref.at[c, slab])

    def load_idx_block(idx2, buf, b, sem):
      return pltpu.make_async_copy(
          idx2.at[pl.ds(row_base + b * K, K)], buf, sem)

    def agg_pass(gidx2, sidx2, table, out_ref):
      init_acc()
      load_idx_block(gidx2, gib.at[0], 0, si).start()
      load_idx_block(sidx2, sib.at[0], 0, si).start()
      load_idx_block(gidx2, gib.at[0], 0, si).wait()
      load_idx_block(sidx2, sib.at[0], 0, si).wait()
      for j in range(K):
        pltpu.async_copy(table.at[gib.at[0, j]], rows[j], sg)

      def superstep(b, p, first, last):
        q = 1 - p
        if not first:
          for j in range(K):
            pltpu.make_async_copy(rows[2 * q + j],
                                  acc.at[sib.at[q, j]], ss).wait()
        if not last:
          load_idx_block(gidx2, gib.at[q], b + 1, si).start()
          load_idx_block(sidx2, sib.at[q], b + 1, si).start()
        for j in range(K):
          pltpu.make_async_copy(table.at[gib.at[p, j]],
                                rows[2 * p + j], sg).wait()
        for j in range(K):
          pltpu.async_copy(rows[2 * p + j], acc.at[sib.at[p, j]], ss,
                           add=True)
        if not last:
          load_idx_block(gidx2, gib.at[q], b + 1, si).wait()
          load_idx_block(sidx2, sib.at[q], b + 1, si).wait()
          for j in range(K):
            pltpu.async_copy(table.at[gib.at[q, j]], rows[2 * q + j], sg)

      superstep(0, 0, True, False)

      def pair(t, carry):
        superstep(2 * t + 1, 1, False, False)
        superstep(2 * t + 2, 0, False, False)
        return carry

      lax.fori_loop(0, (NB - 2) // 2, pair, 0)
      superstep(NB - 1, 1, False, True)
      for j in range(K):
        pltpu.make_async_copy(rows[2 + j], acc.at[sib.at[1, j]], ss).wait()
      dump_acc(out_ref)

    def deg_pass(sidx2, out_ref):
      init_acc()
      load_idx_block(sidx2, sib.at[0], 0, si).start()
      load_idx_block(sidx2, sib.at[0], 0, si).wait()

      def superstep(b, p, first, last):
        q = 1 - p
        if not first:
          for j in range(K):
            pltpu.make_async_copy(r0, acc.at[sib.at[q, j]], ss).wait()
        if not last:
          load_idx_block(sidx2, sib.at[q], b + 1, si).start()
        for j in range(K):
          pltpu.async_copy(r0, acc.at[sib.at[p, j]], ss, add=True)
        if not last:
          load_idx_block(sidx2, sib.at[q], b + 1, si).wait()

      superstep(0, 0, True, False)

      def pair(t, carry):
        superstep(2 * t + 1, 1, False, False)
        superstep(2 * t + 2, 0, False, False)
        return carry

      lax.fori_loop(0, (NB - 2) // 2, pair, 0)
      superstep(NB - 1, 1, False, True)
      for j in range(K):
        pltpu.make_async_copy(r0, acc.at[sib.at[1, j]], ss).wait()
      dump_acc(out_ref)

    agg_pass(src2, dst2, he, aggu_o)
    agg_pass(dst2, src2, hu, agge_o)
    if with_deg:
      pltpu.sync_copy(ones, r0)
      deg_pass(dst2, degu_o)
      deg_pass(src2, dege_o)

  return pl.kernel(body, out_type=out_type, mesh=mesh, scratch_types=scratch,
                   name="sc_agg_deg" if with_deg else "sc_agg")


_sc_agg_deg = _make_sc_agg(True)
_sc_agg = _make_sc_agg(False)


def _matmul(x, w, b):
  y = lax.dot_general(x, w, (((1,), (1,)), ((), ())),
                      preferred_element_type=_f32)
  return y + b


def _proj2_body(xu, wu, bu, xe, we, be, hu_o, he_o):
  hu_o[:NU] = _matmul(xu[:], wu[:], bu[:])
  hu_o[NU:] = jnp.zeros((R - NU, D), _f32)
  he_o[:NEV] = _matmul(xe[:], we[:], be[:])
  he_o[NEV:] = jnp.zeros((R - NEV, D), _f32)


def _norm(aggp, h, degp):
  agg = aggp[0] + aggp[1] + h
  deg = degp[0] + degp[1]
  return agg[:NU] / (deg[:NU, 0:1] + 1.0)


def _combine_proj2_body(aggu, agge, hu, he, degu, dege, wu, bu, we, be,
                        hu_o, he_o):
  xu = _norm(aggu[:], hu[:], degu[:])
  xe = _norm(agge[:], he[:], dege[:])
  hu_o[:NU] = _matmul(xu, wu[:], bu[:])
  hu_o[NU:] = jnp.zeros((R - NU, D), _f32)
  he_o[:NEV] = _matmul(xe, we[:], be[:])
  he_o[NEV:] = jnp.zeros((R - NEV, D), _f32)


def _final2_body(aggu, agge, hu, he, degu, dege, ou, oe):
  ou[...] = _norm(aggu[:], hu[:], degu[:])
  oe[...] = _norm(agge[:], he[:], dege[:])


_proj2 = pl.pallas_call(
    _proj2_body,
    out_shape=(jax.ShapeDtypeStruct((R, D), _f32),
               jax.ShapeDtypeStruct((R, D), _f32)),
)

_combine_proj2 = pl.pallas_call(
    _combine_proj2_body,
    out_shape=(jax.ShapeDtypeStruct((R, D), _f32),
               jax.ShapeDtypeStruct((R, D), _f32)),
)

_final2 = pl.pallas_call(
    _final2_body,
    out_shape=(jax.ShapeDtypeStruct((NU, D), _f32),
               jax.ShapeDtypeStruct((NEV, D), _f32)),
)


@jax.jit
def kernel(x_user, x_event, Wu0, bu0, We0, be0, Wu1, bu1, We1, be1, edge_index):
  ei = edge_index.astype(jnp.int32)
  pad = jnp.full((EP - E,), TRASH, jnp.int32)
  src = jnp.concatenate([ei[0], pad]).reshape(NROW, CH)
  dst = jnp.concatenate([ei[1], pad]).reshape(NROW, CH)

  zeros = jnp.zeros((R, D), _f32)
  ones = jnp.ones((CH, D), _f32)

  bu0r = bu0.reshape(1, D)
  be0r = be0.reshape(1, D)
  bu1r = bu1.reshape(1, D)
  be1r = be1.reshape(1, D)

  hu0, he0 = _proj2(x_user, Wu0, bu0r, x_event, We0, be0r)
  aggu, agge, degu, dege = _sc_agg_deg(he0, hu0, src, dst, zeros, ones)
  hu1, he1 = _combine_proj2(aggu, agge, hu0, he0, degu, dege,
                            Wu1, bu1r, We1, be1r)
  aggu2, agge2 = _sc_agg(he1, hu1, src, dst, zeros)
  return _final2(aggu2, agge2, hu1, he1, degu, dege)


# speedup vs baseline: 1.0926x; 1.0926x over previous
import jax
import jax.numpy as jnp
from jax import lax
from jax.experimental import pallas as pl
from jax.experimental.pallas import tpu as pltpu
from jax.experimental.pallas import tpu_sc as plsc

NU = 5000
NEV = 5000
D = 128
E = 320000
NC = 2
NS = 16
NW = NC * NS
CH = 128
CPT0 = 124
CPT1 = 36
NCHUNK = NS * (CPT0 + CPT1)
EP = NCHUNK * CH
NROW = NCHUNK
K = 2
NB0 = CPT0 // K
NB1 = CPT1 // K
R = 5120
RPT = R // NS
TRASH = 5000

_f32 = jnp.float32


def _make_sc_agg(with_deg):
  mesh = plsc.VectorSubcoreMesh(core_axis_name="c", subcore_axis_name="s")

  out_type = [
      jax.ShapeDtypeStruct((NC, R, D), _f32),
      jax.ShapeDtypeStruct((NC, R, D), _f32),
  ]
  if with_deg:
    out_type += [
        jax.ShapeDtypeStruct((NC, R, D), _f32),
        jax.ShapeDtypeStruct((NC, R, D), _f32),
    ]
  scratch = [
      pltpu.VMEM((2, K, CH), jnp.int32),
      pltpu.VMEM((2, K, CH), jnp.int32),
      pltpu.VMEM((CH, D), _f32),
      pltpu.VMEM((CH, D), _f32),
      pltpu.VMEM((CH, D), _f32),
      pltpu.VMEM((CH, D), _f32),
      pltpu.VMEM_SHARED((R, D), _f32),
      pltpu.SemaphoreType.DMA,
      pltpu.SemaphoreType.DMA,
      pltpu.SemaphoreType.DMA,
  ]

  def body(*refs):
    if with_deg:
      (he, hu, src2, dst2, zeros, ones,
       aggu_o, agge_o, degu_o, dege_o,
       gib, sib, r0, r1, r2, r3, acc, sg, ss, si) = refs
    else:
      (he, hu, src2, dst2, zeros,
       aggu_o, agge_o,
       gib, sib, r0, r1, r2, r3, acc, sg, ss, si) = refs

    rows = (r0, r1, r2, r3)
    c = lax.axis_index("c")
    s = lax.axis_index("s")
    row_base = (1 - c) * (s * CPT0) + c * (NS * CPT0 + s * CPT1)
    nb = (1 - c) * NB0 + c * NB1
    slab = pl.ds(s * RPT, RPT)

    def init_acc():
      pltpu.sync_copy(zeros.at[slab], acc.at[slab])
      plsc.subcore_barrier()

    def dump_acc(out_ref):
      plsc.subcore_barrier()
      pltpu.sync_copy(acc.at[slab], out_ref.at[c, slab])

    def load_idx_block(idx2, buf, b, sem):
      return pltpu.make_async_copy(
          idx2.at[pl.ds(row_base + b * K, K)], buf, sem)

    def agg_pass(gidx2, sidx2, table, out_ref):
      init_acc()
      load_idx_block(gidx2, gib.at[0], 0, si).start()
      load_idx_block(sidx2, sib.at[0], 0, si).start()
      load_idx_block(gidx2, gib.at[0], 0, si).wait()
      load_idx_block(sidx2, sib.at[0], 0, si).wait()
      for j in range(K):
        pltpu.async_copy(table.at[gib.at[0, j]], rows[j], sg)

      def superstep(b, p, first, last):
        q = 1 - p
        if not first:
          for j in range(K):
            pltpu.make_async_copy(rows[2 * q + j],
                                  acc.at[sib.at[q, j]], ss).wait()
        if not last:
          load_idx_block(gidx2, gib.at[q], b + 1, si).start()
          load_idx_block(sidx2, sib.at[q], b + 1, si).start()
        for j in range(K):
          pltpu.make_async_copy(table.at[gib.at[p, j]],
                                rows[2 * p + j], sg).wait()
        for j in range(K):
          pltpu.async_copy(rows[2 * p + j], acc.at[sib.at[p, j]], ss,
                           add=True)
        if not last:
          load_idx_block(gidx2, gib.at[q], b + 1, si).wait()
          load_idx_block(sidx2, sib.at[q], b + 1, si).wait()
          for j in range(K):
            pltpu.async_copy(table.at[gib.at[q, j]], rows[2 * q + j], sg)

      superstep(0, 0, True, False)

      def pair(t, carry):
        superstep(2 * t + 1, 1, False, False)
        superstep(2 * t + 2, 0, False, False)
        return carry

      lax.fori_loop(0, (nb - 2) // 2, pair, 0)
      superstep(nb - 1, 1, False, True)
      for j in range(K):
        pltpu.make_async_copy(rows[2 + j], acc.at[sib.at[1, j]], ss).wait()
      dump_acc(out_ref)

    def deg_pass(sidx2, out_ref):
      init_acc()
      load_idx_block(sidx2, sib.at[0], 0, si).start()
      load_idx_block(sidx2, sib.at[0], 0, si).wait()

      def superstep(b, p, first, last):
        q = 1 - p
        if not first:
          for j in range(K):
            pltpu.make_async_copy(r0, acc.at[sib.at[q, j]], ss).wait()
        if not last:
          load_idx_block(sidx2, sib.at[q], b + 1, si).start()
        for j in range(K):
          pltpu.async_copy(r0, acc.at[sib.at[p, j]], ss, add=True)
        if not last:
          load_idx_block(sidx2, sib.at[q], b + 1, si).wait()

      superstep(0, 0, True, False)

      def pair(t, carry):
        superstep(2 * t + 1, 1, False, False)
        superstep(2 * t + 2, 0, False, False)
        return carry

      lax.fori_loop(0, (nb - 2) // 2, pair, 0)
      superstep(nb - 1, 1, False, True)
      for j in range(K):
        pltpu.make_async_copy(r0, acc.at[sib.at[1, j]], ss).wait()
      dump_acc(out_ref)

    agg_pass(src2, dst2, he, aggu_o)
    agg_pass(dst2, src2, hu, agge_o)
    if with_deg:
      pltpu.sync_copy(ones, r0)
      deg_pass(dst2, degu_o)
      deg_pass(src2, dege_o)

  return pl.kernel(body, out_type=out_type, mesh=mesh, scratch_types=scratch,
                   name="sc_agg_deg" if with_deg else "sc_agg")


_sc_agg_deg = _make_sc_agg(True)
_sc_agg = _make_sc_agg(False)


def _matmul(x, w, b):
  y = lax.dot_general(x, w, (((1,), (1,)), ((), ())),
                      preferred_element_type=_f32)
  return y + b


def _proj2_body(xu, wu, bu, xe, we, be, hu_o, he_o):
  hu_o[:NU] = _matmul(xu[:], wu[:], bu[:])
  hu_o[NU:] = jnp.zeros((R - NU, D), _f32)
  he_o[:NEV] = _matmul(xe[:], we[:], be[:])
  he_o[NEV:] = jnp.zeros((R - NEV, D), _f32)


def _norm(aggp, h, degp):
  agg = aggp[0] + aggp[1] + h
  deg = degp[0] + degp[1]
  return agg[:NU] / (deg[:NU, 0:1] + 1.0)


def _combine_proj2_body(aggu, agge, hu, he, degu, dege, wu, bu, we, be,
                        hu_o, he_o):
  xu = _norm(aggu[:], hu[:], degu[:])
  xe = _norm(agge[:], he[:], dege[:])
  hu_o[:NU] = _matmul(xu, wu[:], bu[:])
  hu_o[NU:] = jnp.zeros((R - NU, D), _f32)
  he_o[:NEV] = _matmul(xe, we[:], be[:])
  he_o[NEV:] = jnp.zeros((R - NEV, D), _f32)


def _final2_body(aggu, agge, hu, he, degu, dege, ou, oe):
  ou[...] = _norm(aggu[:], hu[:], degu[:])
  oe[...] = _norm(agge[:], he[:], dege[:])


_proj2 = pl.pallas_call(
    _proj2_body,
    out_shape=(jax.ShapeDtypeStruct((R, D), _f32),
               jax.ShapeDtypeStruct((R, D), _f32)),
)

_combine_proj2 = pl.pallas_call(
    _combine_proj2_body,
    out_shape=(jax.ShapeDtypeStruct((R, D), _f32),
               jax.ShapeDtypeStruct((R, D), _f32)),
)

_final2 = pl.pallas_call(
    _final2_body,
    out_shape=(jax.ShapeDtypeStruct((NU, D), _f32),
               jax.ShapeDtypeStruct((NEV, D), _f32)),
)


@jax.jit
def kernel(x_user, x_event, Wu0, bu0, We0, be0, Wu1, bu1, We1, be1, edge_index):
  ei = edge_index.astype(jnp.int32)
  pad = jnp.full((EP - E,), TRASH, jnp.int32)
  src = jnp.concatenate([ei[0], pad]).reshape(NROW, CH)
  dst = jnp.concatenate([ei[1], pad]).reshape(NROW, CH)

  zeros = jnp.zeros((R, D), _f32)
  ones = jnp.ones((CH, D), _f32)

  bu0r = bu0.reshape(1, D)
  be0r = be0.reshape(1, D)
  bu1r = bu1.reshape(1, D)
  be1r = be1.reshape(1, D)

  hu0, he0 = _proj2(x_user, Wu0, bu0r, x_event, We0, be0r)
  aggu, agge, degu, dege = _sc_agg_deg(he0, hu0, src, dst, zeros, ones)
  hu1, he1 = _combine_proj2(aggu, agge, hu0, he0, degu, dege,
                            Wu1, bu1r, We1, be1r)
  aggu2, agge2 = _sc_agg(he1, hu1, src, dst, zeros)
  return _final2(aggu2, agge2, hu1, he1, degu, dege)


# speedup vs baseline: 1.0932x; 1.0006x over previous
import jax
import jax.numpy as jnp
from jax import lax
from jax.experimental import pallas as pl
from jax.experimental.pallas import tpu as pltpu
from jax.experimental.pallas import tpu_sc as plsc

NU = 5000
NEV = 5000
D = 128
E = 320000
NC = 2
NS = 16
NW = NC * NS
CH = 128
CPT0 = 124
CPT1 = 36
NCHUNK = NS * (CPT0 + CPT1)
EP = NCHUNK * CH
NROW = NCHUNK
K = 2
NB0 = CPT0 // K
NB1 = CPT1 // K
R = 5120
RPT = R // NS
TRASH = 5000

_f32 = jnp.float32


def _make_sc_agg(with_deg):
  mesh = plsc.VectorSubcoreMesh(core_axis_name="c", subcore_axis_name="s")

  out_type = [
      jax.ShapeDtypeStruct((NC, R, D), _f32),
      jax.ShapeDtypeStruct((NC, R, D), _f32),
  ]
  if with_deg:
    out_type += [
        jax.ShapeDtypeStruct((NC, R, D), _f32),
        jax.ShapeDtypeStruct((NC, R, D), _f32),
    ]
  scratch = [
      pltpu.VMEM((2, K, CH), jnp.int32),
      pltpu.VMEM((2, K, CH), jnp.int32),
      pltpu.VMEM((CH, D), _f32),
      pltpu.VMEM((CH, D), _f32),
      pltpu.VMEM((CH, D), _f32),
      pltpu.VMEM((CH, D), _f32),
      pltpu.VMEM_SHARED((R, D), _f32),
      pltpu.SemaphoreType.DMA,
      pltpu.SemaphoreType.DMA,
      pltpu.SemaphoreType.DMA,
  ]

  def body(*refs):
    if with_deg:
      (he, hu, src2, dst2, zeros, ones,
       aggu_o, agge_o, degu_o, dege_o,
       gib, sib, r0, r1, r2, r3, acc, sg, ss, si) = refs
    else:
      (he, hu, src2, dst2, zeros,
       aggu_o, agge_o,
       gib, sib, r0, r1, r2, r3, acc, sg, ss, si) = refs

    rows = (r0, r1, r2, r3)
    c = lax.axis_index("c")
    s = lax.axis_index("s")
    row_base = (1 - c) * (s * CPT0) + c * (NS * CPT0 + s * CPT1)
    nb = (1 - c) * NB0 + c * NB1
    slab = pl.ds(s * RPT, RPT)

    def init_acc():
      with jax.named_scope("acc_init"):
        pltpu.sync_copy(zeros.at[slab], acc.at[slab])
        plsc.subcore_barrier()

    def dump_acc(out_ref):
      with jax.named_scope("acc_dump"):
        plsc.subcore_barrier()
        pltpu.sync_copy(acc.at[slab], out_ref.at[c, slab])

    def load_idx_block(idx2, buf, b, sem):
      return pltpu.make_async_copy(
          idx2.at[pl.ds(row_base + b * K, K)], buf, sem)

    def agg_pass(gidx2, sidx2, table, out_ref):
      init_acc()
      load_idx_block(gidx2, gib.at[0], 0, si).start()
      load_idx_block(sidx2, sib.at[0], 0, si).start()
      load_idx_block(gidx2, gib.at[0], 0, si).wait()
      load_idx_block(sidx2, sib.at[0], 0, si).wait()
      for j in range(K):
        pltpu.async_copy(table.at[gib.at[0, j]], rows[j], sg)

      def superstep(b, p, first, last):
        q = 1 - p
        if not first:
          for j in range(K):
            pltpu.make_async_copy(rows[2 * q + j],
                                  acc.at[sib.at[q, j]], ss).wait()
        if not last:
          load_idx_block(gidx2, gib.at[q], b + 1, si).start()
          load_idx_block(sidx2, sib.at[q], b + 1, si).start()
        for j in range(K):
          pltpu.make_async_copy(table.at[gib.at[p, j]],
                                rows[2 * p + j], sg).wait()
        for j in range(K):
          pltpu.async_copy(rows[2 * p + j], acc.at[sib.at[p, j]], ss,
                           add=True)
        if not last:
          load_idx_block(gidx2, gib.at[q], b + 1, si).wait()
          load_idx_block(sidx2, sib.at[q], b + 1, si).wait()
          for j in range(K):
            pltpu.async_copy(table.at[gib.at[q, j]], rows[2 * q + j], sg)

      with jax.named_scope("agg_pipe"):
        superstep(0, 0, True, False)

        def pair(t, carry):
          superstep(2 * t + 1, 1, False, False)
          superstep(2 * t + 2, 0, False, False)
          return carry

        lax.fori_loop(0, (nb - 2) // 2, pair, 0)
        superstep(nb - 1, 1, False, True)
        for j in range(K):
          pltpu.make_async_copy(rows[2 + j], acc.at[sib.at[1, j]], ss).wait()
      dump_acc(out_ref)

    def deg_pass(sidx2, out_ref):
      init_acc()
      load_idx_block(sidx2, sib.at[0], 0, si).start()
      load_idx_block(sidx2, sib.at[0], 0, si).wait()

      def superstep(b, p, first, last):
        q = 1 - p
        if not first:
          for j in range(K):
            pltpu.make_async_copy(r0, acc.at[sib.at[q, j]], ss).wait()
        if not last:
          load_idx_block(sidx2, sib.at[q], b + 1, si).start()
        for j in range(K):
          pltpu.async_copy(r0, acc.at[sib.at[p, j]], ss, add=True)
        if not last:
          load_idx_block(sidx2, sib.at[q], b + 1, si).wait()

      superstep(0, 0, True, False)

      def pair(t, carry):
        superstep(2 * t + 1, 1, False, False)
        superstep(2 * t + 2, 0, False, False)
        return carry

      lax.fori_loop(0, (nb - 2) // 2, pair, 0)
      superstep(nb - 1, 1, False, True)
      for j in range(K):
        pltpu.make_async_copy(r0, acc.at[sib.at[1, j]], ss).wait()
      dump_acc(out_ref)

    agg_pass(src2, dst2, he, aggu_o)
    agg_pass(dst2, src2, hu, agge_o)
    if with_deg:
      pltpu.sync_copy(ones, r0)
      deg_pass(dst2, degu_o)
      deg_pass(src2, dege_o)

  return pl.kernel(body, out_type=out_type, mesh=mesh, scratch_types=scratch,
                   name="sc_agg_deg" if with_deg else "sc_agg")


_sc_agg_deg = _make_sc_agg(True)
_sc_agg = _make_sc_agg(False)


def _matmul(x, w, b):
  y = lax.dot_general(x, w, (((1,), (1,)), ((), ())),
                      preferred_element_type=_f32)
  return y + b


def _proj2_body(xu, wu, bu, xe, we, be, hu_o, he_o):
  hu_o[:NU] = _matmul(xu[:], wu[:], bu[:])
  hu_o[NU:] = jnp.zeros((R - NU, D), _f32)
  he_o[:NEV] = _matmul(xe[:], we[:], be[:])
  he_o[NEV:] = jnp.zeros((R - NEV, D), _f32)


def _norm(aggp, h, degp):
  agg = aggp[0] + aggp[1] + h
  deg = degp[0] + degp[1]
  return agg[:NU] / (deg[:NU, 0:1] + 1.0)


def _combine_proj2_body(aggu, agge, hu, he, degu, dege, wu, bu, we, be,
                        hu_o, he_o):
  xu = _norm(aggu[:], hu[:], degu[:])
  xe = _norm(agge[:], he[:], dege[:])
  hu_o[:NU] = _matmul(xu, wu[:], bu[:])
  hu_o[NU:] = jnp.zeros((R - NU, D), _f32)
  he_o[:NEV] = _matmul(xe, we[:], be[:])
  he_o[NEV:] = jnp.zeros((R - NEV, D), _f32)


def _final2_body(aggu, agge, hu, he, degu, dege, ou, oe):
  ou[...] = _norm(aggu[:], hu[:], degu[:])
  oe[...] = _norm(agge[:], he[:], dege[:])


_proj2 = pl.pallas_call(
    _proj2_body,
    out_shape=(jax.ShapeDtypeStruct((R, D), _f32),
               jax.ShapeDtypeStruct((R, D), _f32)),
)

_combine_proj2 = pl.pallas_call(
    _combine_proj2_body,
    out_shape=(jax.ShapeDtypeStruct((R, D), _f32),
               jax.ShapeDtypeStruct((R, D), _f32)),
)

_final2 = pl.pallas_call(
    _final2_body,
    out_shape=(jax.ShapeDtypeStruct((NU, D), _f32),
               jax.ShapeDtypeStruct((NEV, D), _f32)),
)


@jax.jit
def kernel(x_user, x_event, Wu0, bu0, We0, be0, Wu1, bu1, We1, be1, edge_index):
  ei = edge_index.astype(jnp.int32)
  pad = jnp.full((EP - E,), TRASH, jnp.int32)
  src = jnp.concatenate([ei[0], pad]).reshape(NROW, CH)
  dst = jnp.concatenate([ei[1], pad]).reshape(NROW, CH)

  zeros = jnp.zeros((R, D), _f32)
  ones = jnp.ones((CH, D), _f32)

  bu0r = bu0.reshape(1, D)
  be0r = be0.reshape(1, D)
  bu1r = bu1.reshape(1, D)
  be1r = be1.reshape(1, D)

  hu0, he0 = _proj2(x_user, Wu0, bu0r, x_event, We0, be0r)
  aggu, agge, degu, dege = _sc_agg_deg(he0, hu0, src, dst, zeros, ones)
  hu1, he1 = _combine_proj2(aggu, agge, hu0, he0, degu, dege,
                            Wu1, bu1r, We1, be1r)
  aggu2, agge2 = _sc_agg(he1, hu1, src, dst, zeros)
  return _final2(aggu2, agge2, hu1, he1, degu, dege)
